# baseline (device time: 136177 ns/iter reference)
import jax
import jax.numpy as jnp
from jax import lax
from jax.experimental import pallas as pl
from jax.experimental.pallas import tpu as pltpu

CH = 256
NCH = 32
T = [c % 3 for c in range(NCH)]


def kernel(x):
    m_per, n = x.shape

    mx = lax.axis_index("x")
    mz = lax.axis_index("z")
    i_tab = jnp.array([[0, 2, 2, 0], [1, 0, 0, 1]], jnp.int32)
    i_my = i_tab[mx, mz]
    i_xn = i_tab[1 - mx, mz]
    i_miss = 3 - i_my - i_xn
    s_z = jnp.array([1, 2, 2, 1], jnp.int32)[mz]
    routing = jnp.stack([i_my, i_xn, i_miss, s_z])

    def body(r_ref, x_hbm, out_ref, f32_buf, load_sems,
             ysend, yrecv, xsend, xrecv, zsend, zrecv):
        my_x = lax.axis_index("x")
        my_y = lax.axis_index("y")
        my_z = lax.axis_index("z")
        i_my, i_xn, i_miss, s_z = r_ref[0], r_ref[1], r_ref[2], r_ref[3]
        y_nbr = (my_x, 1 - my_y, my_z)
        x_nbr = (1 - my_x, my_y, my_z)
        z_nbr = (my_x, my_y, my_z + 1 - 2 * (my_z % 2))

        barrier_sem = pltpu.get_barrier_semaphore()
        for nbr in (y_nbr, x_nbr, z_nbr):
            pl.semaphore_signal(
                barrier_sem, inc=1,
                device_id=nbr, device_id_type=pl.DeviceIdType.MESH,
            )
        pl.semaphore_wait(barrier_sem, 3)

        own_base = my_y * m_per
        opp_base = (1 - my_y) * m_per

        def rdma(base, c, sem_s, sem_r, dev):
            rows = pl.ds(base + c * CH, CH)
            return pltpu.make_async_remote_copy(
                src_ref=out_ref.at[rows, :],
                dst_ref=out_ref.at[rows, :],
                send_sem=sem_s.at[c],
                recv_sem=sem_r.at[c],
                device_id=dev,
                device_id_type=pl.DeviceIdType.MESH,
            )

        def load(j, slot):
            return pltpu.make_async_copy(
                x_hbm.at[pl.ds(j * CH, CH), :],
                f32_buf.at[slot],
                load_sems.at[slot],
            )

        load(0, 0).start()
        for j in range(NCH):
            slot = j % 2
            if j + 1 < NCH:
                load(j + 1, (j + 1) % 2).start()
            load(j, slot).wait()
            out_ref[pl.ds(own_base + j * CH, CH), :] = (
                f32_buf[slot].astype(jnp.bfloat16)
            )

            @pl.when(i_my == T[j])
            def _():
                rdma(own_base, j, ysend, yrecv, y_nbr).start()

        for k in range(NCH + 2):
            if k < NCH:
                c = k

                @pl.when(i_my == T[c])
                def _():
                    rdma(opp_base, c, ysend, yrecv, y_nbr).wait_recv()
                    rdma(opp_base, c, xsend, xrecv, x_nbr).start()

                @pl.when(jnp.logical_and(s_z == T[c], i_my == T[c]))
                def _():
                    rdma(opp_base, c, zsend, zrecv, z_nbr).start()

            if k >= 2:
                d = k - 2

                @pl.when(i_xn == T[d])
                def _():
                    rdma(opp_base, d, xsend, xrecv, x_nbr).wait_recv()

                @pl.when(jnp.logical_and(s_z == T[d], i_xn == T[d]))
                def _():
                    rdma(opp_base, d, zsend, zrecv, z_nbr).start()

        for c in range(NCH):
            @pl.when(i_miss == T[c])
            def _():
                rdma(opp_base, c, zsend, zrecv, z_nbr).wait_recv()

            @pl.when(i_my == T[c])
            def _():
                rdma(own_base, c, ysend, yrecv, y_nbr).wait_send()
                rdma(opp_base, c, xsend, xrecv, x_nbr).wait_send()

            @pl.when(s_z == T[c])
            def _():
                rdma(opp_base, c, zsend, zrecv, z_nbr).wait_send()

    return pl.pallas_call(
        body,
        out_shape=jax.ShapeDtypeStruct((2 * m_per, n), jnp.bfloat16),
        in_specs=[
            pl.BlockSpec(memory_space=pltpu.SMEM),
            pl.BlockSpec(memory_space=pl.ANY),
        ],
        out_specs=pl.BlockSpec(memory_space=pltpu.VMEM),
        scratch_shapes=[
            pltpu.VMEM((2, CH, n), jnp.float32),
            pltpu.SemaphoreType.DMA((2,)),
            pltpu.SemaphoreType.DMA((NCH,)),
            pltpu.SemaphoreType.DMA((NCH,)),
            pltpu.SemaphoreType.DMA((NCH,)),
            pltpu.SemaphoreType.DMA((NCH,)),
            pltpu.SemaphoreType.DMA((NCH,)),
            pltpu.SemaphoreType.DMA((NCH,)),
        ],
        compiler_params=pltpu.CompilerParams(
            collective_id=0,
            vmem_limit_bytes=56 * 1024 * 1024,
        ),
    )(routing, x)


# device time: 118300 ns/iter; 1.1511x vs baseline; 1.1511x over previous
import jax

jax.config.update("jax_compilation_cache_dir", "/tmp/scband_jax_cache")
jax.config.update("jax_persistent_cache_min_compile_time_secs", 1)

import jax.numpy as jnp
from jax import lax
from jax.experimental import pallas as pl
from jax.experimental.pallas import tpu as pltpu

CH = 256
NCH = 32
J = 11


def kernel(x):
    m_per, n = x.shape

    def body(x_hbm, out_ref, f32_buf, load_sems,
             ysend, yrecv, xsend, xrecv, zsend, zrecv):
        my_x = lax.axis_index("x")
        my_y = lax.axis_index("y")
        my_z = lax.axis_index("z")
        ze = jnp.logical_or(my_z == 0, my_z == 3)
        i_my = jnp.where(my_x == 0, jnp.where(ze, 0, 2), jnp.where(ze, 1, 0))
        i_xn = jnp.where(my_x == 0, jnp.where(ze, 1, 0), jnp.where(ze, 0, 2))
        i_miss = 3 - i_my - i_xn
        s_z = jnp.where(ze, 1, 2)
        t_a = lax.rem(i_my + 1, 3)
        t_b = lax.rem(i_my + 2, 3)

        y_nbr = (my_x, 1 - my_y, my_z)
        x_nbr = (1 - my_x, my_y, my_z)
        z_nbr = (my_x, my_y, my_z + 1 - 2 * (my_z % 2))

        barrier_sem = pltpu.get_barrier_semaphore()
        for nbr in (y_nbr, x_nbr, z_nbr):
            pl.semaphore_signal(
                barrier_sem, inc=1,
                device_id=nbr, device_id_type=pl.DeviceIdType.MESH,
            )
        pl.semaphore_wait(barrier_sem, 3)

        own_base = my_y * m_per
        opp_base = (1 - my_y) * m_per

        def when(pred, fn):
            if pred is None:
                fn()
            else:
                pl.when(pred)(fn)

        def rdma(base, c_expr, sems_s, sems_r, j, dev):
            rows = pl.ds(base + c_expr * CH, CH)
            return pltpu.make_async_remote_copy(
                src_ref=out_ref.at[rows, :],
                dst_ref=out_ref.at[rows, :],
                send_sem=sems_s.at[j],
                recv_sem=sems_r.at[j],
                device_id=dev,
                device_id_type=pl.DeviceIdType.MESH,
            )

        seq = [(3 * j + i_my, None if j < 10 else (i_my <= 1))
               for j in range(J)]
        for j in range(J):
            seq.append((3 * j + t_a, None if j < 10 else (t_a <= 1)))
            seq.append((3 * j + t_b, None if j < 10 else (t_b <= 1)))

        def load_desc(i):
            c_expr, _ = seq[i]
            return pltpu.make_async_copy(
                x_hbm.at[pl.ds(c_expr * CH, CH), :],
                f32_buf.at[i % 2],
                load_sems.at[i % 2],
            )

        li = [0]

        def step_load():
            i = li[0]
            li[0] = i + 1
            if i + 1 < len(seq):
                when(seq[i + 1][1], lambda: load_desc(i + 1).start())
            c_expr, pred = seq[i]

            def fin():
                load_desc(i).wait()
                out_ref[pl.ds(own_base + c_expr * CH, CH), :] = (
                    f32_buf[i % 2].astype(jnp.bfloat16)
                )
            when(pred, fin)

        tail_my = i_my <= 1
        tail_xn = i_xn <= 1
        tail_zs = s_z <= 1
        tail_miss = i_miss <= 1

        def p(j, tail):
            return None if j < 10 else tail

        def land(pa, pb):
            if pa is None:
                return pb
            if pb is None:
                return pa
            return jnp.logical_and(pa, pb)

        when(seq[0][1], lambda: load_desc(0).start())
        for j in range(J):
            step_load()
            when(p(j, tail_my),
                 lambda j=j: rdma(own_base, 3 * j + i_my,
                                  ysend, yrecv, j, y_nbr).start())

        for j in range(J):
            step_load()
            step_load()

            def fwd(j=j):
                rdma(opp_base, 3 * j + i_my, ysend, yrecv, j, y_nbr).wait_recv()
                rdma(opp_base, 3 * j + i_my, xsend, xrecv, j, x_nbr).start()
            when(p(j, tail_my), fwd)
            when(land(p(j, tail_my), s_z == i_my),
                 lambda j=j: rdma(opp_base, 3 * j + s_z,
                                  zsend, zrecv, j, z_nbr).start())

            if j >= 2:
                d = j - 2
                when(p(d, tail_xn),
                     lambda d=d: rdma(opp_base, 3 * d + i_xn,
                                      xsend, xrecv, d, x_nbr).wait_recv())
                when(land(p(d, tail_xn), s_z == i_xn),
                     lambda d=d: rdma(opp_base, 3 * d + s_z,
                                      zsend, zrecv, d, z_nbr).start())

        for d in (J - 2, J - 1):
            when(p(d, tail_xn),
                 lambda d=d: rdma(opp_base, 3 * d + i_xn,
                                  xsend, xrecv, d, x_nbr).wait_recv())
            when(land(p(d, tail_xn), s_z == i_xn),
                 lambda d=d: rdma(opp_base, 3 * d + s_z,
                                  zsend, zrecv, d, z_nbr).start())
        for j in range(J):
            when(p(j, tail_miss),
                 lambda j=j: rdma(opp_base, 3 * j + i_miss,
                                  zsend, zrecv, j, z_nbr).wait_recv())
        for j in range(J):
            when(p(j, tail_my),
                 lambda j=j: rdma(own_base, 3 * j + i_my,
                                  ysend, yrecv, j, y_nbr).wait_send())
            when(p(j, tail_my),
                 lambda j=j: rdma(opp_base, 3 * j + i_my,
                                  xsend, xrecv, j, x_nbr).wait_send())
            when(p(j, tail_zs),
                 lambda j=j: rdma(opp_base, 3 * j + s_z,
                                  zsend, zrecv, j, z_nbr).wait_send())

    return pl.pallas_call(
        body,
        out_shape=jax.ShapeDtypeStruct((2 * m_per, n), jnp.bfloat16),
        in_specs=[pl.BlockSpec(memory_space=pl.ANY)],
        out_specs=pl.BlockSpec(memory_space=pltpu.VMEM),
        scratch_shapes=[
            pltpu.VMEM((2, CH, n), jnp.float32),
            pltpu.SemaphoreType.DMA((2,)),
            pltpu.SemaphoreType.DMA((J,)),
            pltpu.SemaphoreType.DMA((J,)),
            pltpu.SemaphoreType.DMA((J,)),
            pltpu.SemaphoreType.DMA((J,)),
            pltpu.SemaphoreType.DMA((J,)),
            pltpu.SemaphoreType.DMA((J,)),
        ],
        compiler_params=pltpu.CompilerParams(
            collective_id=0,
            vmem_limit_bytes=56 * 1024 * 1024,
        ),
    )(x)


# device time: 108700 ns/iter; 1.2528x vs baseline; 1.0883x over previous
import jax

jax.config.update("jax_compilation_cache_dir", "/tmp/scband_jax_cache")
jax.config.update("jax_persistent_cache_min_compile_time_secs", 1)

import jax.numpy as jnp
from jax import lax
from jax.experimental import pallas as pl
from jax.experimental.pallas import tpu as pltpu

CH = 256
NCH = 32
J = 11


def kernel(x):
    m_per, n = x.shape

    def body(x_hbm, out_ref, mine_buf, f32_buf, stage, load_sems, store_sems,
             msem, ysend, yrecv, xsend, xrecv, zsend, zrecv):
        my_x = lax.axis_index("x")
        my_y = lax.axis_index("y")
        my_z = lax.axis_index("z")
        ze = jnp.logical_or(my_z == 0, my_z == 3)
        i_my = jnp.where(my_x == 0, jnp.where(ze, 0, 2), jnp.where(ze, 1, 0))
        i_xn = jnp.where(my_x == 0, jnp.where(ze, 1, 0), jnp.where(ze, 0, 2))
        i_miss = 3 - i_my - i_xn
        s_z = jnp.where(ze, 1, 2)
        t_a = lax.rem(i_my + 1, 3)
        t_b = lax.rem(i_my + 2, 3)

        y_nbr = (my_x, 1 - my_y, my_z)
        x_nbr = (1 - my_x, my_y, my_z)
        z_nbr = (my_x, my_y, my_z + 1 - 2 * (my_z % 2))

        barrier_sem = pltpu.get_barrier_semaphore()
        for nbr in (y_nbr, x_nbr, z_nbr):
            pl.semaphore_signal(
                barrier_sem, inc=1,
                device_id=nbr, device_id_type=pl.DeviceIdType.MESH,
            )
        pl.semaphore_wait(barrier_sem, 3)

        own_base = my_y * m_per
        opp_base = (1 - my_y) * m_per

        def when(pred, fn):
            if pred is None:
                fn()
            else:
                pl.when(pred)(fn)

        def rdma(base, c_expr, sems_s, sems_r, j, dev, src=None):
            rows = pl.ds(base + c_expr * CH, CH)
            return pltpu.make_async_remote_copy(
                src_ref=out_ref.at[rows, :] if src is None else src,
                dst_ref=out_ref.at[rows, :],
                send_sem=sems_s.at[j],
                recv_sem=sems_r.at[j],
                device_id=dev,
                device_id_type=pl.DeviceIdType.MESH,
            )

        def mine_rows(j):
            return pl.ds(j * CH, CH)

        seq = [(3 * j + i_my, None if j < 10 else (i_my <= 1))
               for j in range(J)]
        for j in range(J):
            seq.append((3 * j + t_a, None if j < 10 else (t_a <= 1)))
            seq.append((3 * j + t_b, None if j < 10 else (t_b <= 1)))

        def load_desc(i):
            c_expr, _ = seq[i]
            return pltpu.make_async_copy(
                x_hbm.at[pl.ds(c_expr * CH, CH), :],
                f32_buf.at[i % 2],
                load_sems.at[i % 2],
            )

        li = [0]
        stage_uses = [0]

        def step_load():
            i = li[0]
            li[0] = i + 1
            if i + 1 < len(seq):
                when(seq[i + 1][1], lambda: load_desc(i + 1).start())
            c_expr, pred = seq[i]

            if i < J:
                def fin():
                    load_desc(i).wait()
                    mine_buf[mine_rows(i), :] = f32_buf[i % 2].astype(jnp.bfloat16)
                    pltpu.make_async_copy(
                        mine_buf.at[mine_rows(i), :],
                        out_ref.at[pl.ds(own_base + c_expr * CH, CH), :],
                        msem.at[i],
                    ).start()
            else:
                u = stage_uses[0]
                stage_uses[0] = u + 1
                slot = u % 2

                def fin():
                    load_desc(i).wait()
                    if u >= 2:
                        pltpu.make_async_copy(
                            stage.at[slot], stage.at[slot], store_sems.at[slot]
                        ).wait()
                    stage[slot] = f32_buf[i % 2].astype(jnp.bfloat16)
                    pltpu.make_async_copy(
                        stage.at[slot],
                        out_ref.at[pl.ds(own_base + c_expr * CH, CH), :],
                        store_sems.at[slot],
                    ).start()
            when(pred, fin)

        tail_my = i_my <= 1
        tail_xn = i_xn <= 1
        tail_zs = s_z <= 1
        tail_miss = i_miss <= 1

        def p(j, tail):
            return None if j < 10 else tail

        def land(pa, pb):
            if pa is None:
                return pb
            if pb is None:
                return pa
            return jnp.logical_and(pa, pb)

        when(seq[0][1], lambda: load_desc(0).start())
        for j in range(J):
            step_load()
            when(p(j, tail_my),
                 lambda j=j: rdma(own_base, 3 * j + i_my, ysend, yrecv, j,
                                  y_nbr, src=mine_buf.at[mine_rows(j), :]).start())

        for j in range(J):
            step_load()
            step_load()

            def fwd(j=j):
                rdma(opp_base, 3 * j + i_my, ysend, yrecv, j, y_nbr).wait_recv()
                rdma(opp_base, 3 * j + i_my, xsend, xrecv, j, x_nbr).start()
            when(p(j, tail_my), fwd)
            when(land(p(j, tail_my), s_z == i_my),
                 lambda j=j: rdma(opp_base, 3 * j + s_z,
                                  zsend, zrecv, j, z_nbr).start())

            if j >= 2:
                d = j - 2
                when(p(d, tail_xn),
                     lambda d=d: rdma(opp_base, 3 * d + i_xn,
                                      xsend, xrecv, d, x_nbr).wait_recv())
                when(land(p(d, tail_xn), s_z == i_xn),
                     lambda d=d: rdma(opp_base, 3 * d + s_z,
                                      zsend, zrecv, d, z_nbr).start())

        for d in (J - 2, J - 1):
            when(p(d, tail_xn),
                 lambda d=d: rdma(opp_base, 3 * d + i_xn,
                                  xsend, xrecv, d, x_nbr).wait_recv())
            when(land(p(d, tail_xn), s_z == i_xn),
                 lambda d=d: rdma(opp_base, 3 * d + s_z,
                                  zsend, zrecv, d, z_nbr).start())
        for j in range(J):
            when(p(j, tail_miss),
                 lambda j=j: rdma(opp_base, 3 * j + i_miss,
                                  zsend, zrecv, j, z_nbr).wait_recv())
        for j in range(J):
            when(p(j, tail_my),
                 lambda j=j: rdma(own_base, 3 * j + i_my, ysend, yrecv, j,
                                  y_nbr, src=mine_buf.at[mine_rows(j), :]).wait_send())
            when(p(j, tail_my),
                 lambda j=j: rdma(opp_base, 3 * j + i_my,
                                  xsend, xrecv, j, x_nbr).wait_send())
            when(p(j, tail_zs),
                 lambda j=j: rdma(opp_base, 3 * j + s_z,
                                  zsend, zrecv, j, z_nbr).wait_send())
            when(seq[j][1],
                 lambda j=j: pltpu.make_async_copy(
                     mine_buf.at[mine_rows(j), :],
                     out_ref.at[mine_rows(j), :],
                     msem.at[j],
                 ).wait())
        for slot in range(2):
            if stage_uses[0] > slot:
                pltpu.make_async_copy(
                    stage.at[slot], stage.at[slot], store_sems.at[slot]
                ).wait()

    return pl.pallas_call(
        body,
        out_shape=jax.ShapeDtypeStruct((2 * m_per, n), jnp.bfloat16),
        in_specs=[pl.BlockSpec(memory_space=pl.ANY)],
        out_specs=pl.BlockSpec(memory_space=pl.ANY),
        scratch_shapes=[
            pltpu.VMEM((J * CH, n), jnp.bfloat16),
            pltpu.VMEM((2, CH, n), jnp.float32),
            pltpu.VMEM((2, CH, n), jnp.bfloat16),
            pltpu.SemaphoreType.DMA((2,)),
            pltpu.SemaphoreType.DMA((2,)),
            pltpu.SemaphoreType.DMA((J,)),
            pltpu.SemaphoreType.DMA((J,)),
            pltpu.SemaphoreType.DMA((J,)),
            pltpu.SemaphoreType.DMA((J,)),
            pltpu.SemaphoreType.DMA((J,)),
            pltpu.SemaphoreType.DMA((J,)),
            pltpu.SemaphoreType.DMA((J,)),
        ],
        compiler_params=pltpu.CompilerParams(
            collective_id=0,
            vmem_limit_bytes=56 * 1024 * 1024,
        ),
    )(x)


# device time: 107839 ns/iter; 1.2628x vs baseline; 1.0080x over previous
import jax

jax.config.update("jax_compilation_cache_dir", "/tmp/scband_jax_cache")
jax.config.update("jax_persistent_cache_min_compile_time_secs", 1)

import jax.numpy as jnp
from jax import lax
from jax.experimental import pallas as pl
from jax.experimental.pallas import tpu as pltpu

CH = 128
NCH = 8192 // CH
J = (NCH + 2) // 3
TAIL_BOUND = NCH - 3 * (J - 1) - 1


def kernel(x):
    m_per, n = x.shape

    def body(x_hbm, out_ref, mine_buf, f32_buf, stage, load_sems, store_sems,
             msem, ysend, yrecv, xsend, xrecv, zsend, zrecv):
        my_x = lax.axis_index("x")
        my_y = lax.axis_index("y")
        my_z = lax.axis_index("z")
        ze = jnp.logical_or(my_z == 0, my_z == 3)
        i_my = jnp.where(my_x == 0, jnp.where(ze, 0, 2), jnp.where(ze, 1, 0))
        i_xn = jnp.where(my_x == 0, jnp.where(ze, 1, 0), jnp.where(ze, 0, 2))
        i_miss = 3 - i_my - i_xn
        s_z = jnp.where(ze, 1, 2)
        t_a = lax.rem(i_my + 1, 3)
        t_b = lax.rem(i_my + 2, 3)

        y_nbr = (my_x, 1 - my_y, my_z)
        x_nbr = (1 - my_x, my_y, my_z)
        z_nbr = (my_x, my_y, my_z + 1 - 2 * (my_z % 2))

        barrier_sem = pltpu.get_barrier_semaphore()
        for nbr in (y_nbr, x_nbr, z_nbr):
            pl.semaphore_signal(
                barrier_sem, inc=1,
                device_id=nbr, device_id_type=pl.DeviceIdType.MESH,
            )
        pl.semaphore_wait(barrier_sem, 3)

        own_base = my_y * m_per
        opp_base = (1 - my_y) * m_per

        def when(pred, fn):
            if pred is None:
                fn()
            else:
                pl.when(pred)(fn)

        def rdma(base, c_expr, sems_s, sems_r, j, dev, src=None):
            rows = pl.ds(base + c_expr * CH, CH)
            return pltpu.make_async_remote_copy(
                src_ref=out_ref.at[rows, :] if src is None else src,
                dst_ref=out_ref.at[rows, :],
                send_sem=sems_s.at[j],
                recv_sem=sems_r.at[j],
                device_id=dev,
                device_id_type=pl.DeviceIdType.MESH,
            )

        def mine_rows(j):
            return pl.ds(j * CH, CH)

        seq = [(3 * j + i_my, None if j < J - 1 else (i_my <= TAIL_BOUND))
               for j in range(J)]
        for j in range(J):
            seq.append((3 * j + t_a, None if j < J - 1 else (t_a <= TAIL_BOUND)))
            seq.append((3 * j + t_b, None if j < J - 1 else (t_b <= TAIL_BOUND)))

        def load_desc(i):
            c_expr, _ = seq[i]
            return pltpu.make_async_copy(
                x_hbm.at[pl.ds(c_expr * CH, CH), :],
                f32_buf.at[i % 2],
                load_sems.at[i % 2],
            )

        li = [0]
        stage_uses = [0]

        def step_load():
            i = li[0]
            li[0] = i + 1
            if i + 1 < len(seq):
                when(seq[i + 1][1], lambda: load_desc(i + 1).start())
            c_expr, pred = seq[i]

            if i < J:
                def fin():
                    load_desc(i).wait()
                    mine_buf[mine_rows(i), :] = f32_buf[i % 2].astype(jnp.bfloat16)
                    pltpu.make_async_copy(
                        mine_buf.at[mine_rows(i), :],
                        out_ref.at[pl.ds(own_base + c_expr * CH, CH), :],
                        msem.at[i],
                    ).start()
            else:
                u = stage_uses[0]
                stage_uses[0] = u + 1
                slot = u % 2

                def fin():
                    load_desc(i).wait()
                    if u >= 2:
                        pltpu.make_async_copy(
                            stage.at[slot], stage.at[slot], store_sems.at[slot]
                        ).wait()
                    stage[slot] = f32_buf[i % 2].astype(jnp.bfloat16)
                    pltpu.make_async_copy(
                        stage.at[slot],
                        out_ref.at[pl.ds(own_base + c_expr * CH, CH), :],
                        store_sems.at[slot],
                    ).start()
            when(pred, fin)

        tail_my = i_my <= TAIL_BOUND
        tail_xn = i_xn <= TAIL_BOUND
        tail_zs = s_z <= TAIL_BOUND
        tail_miss = i_miss <= TAIL_BOUND

        def p(j, tail):
            return None if j < J - 1 else tail

        def land(pa, pb):
            if pa is None:
                return pb
            if pb is None:
                return pa
            return jnp.logical_and(pa, pb)

        when(seq[0][1], lambda: load_desc(0).start())
        for j in range(J):
            step_load()
            when(p(j, tail_my),
                 lambda j=j: rdma(own_base, 3 * j + i_my, ysend, yrecv, j,
                                  y_nbr, src=mine_buf.at[mine_rows(j), :]).start())

        for j in range(J):
            step_load()
            step_load()

            def fwd(j=j):
                rdma(opp_base, 3 * j + i_my, ysend, yrecv, j, y_nbr).wait_recv()
                rdma(opp_base, 3 * j + i_my, xsend, xrecv, j, x_nbr).start()
            when(p(j, tail_my), fwd)
            when(land(p(j, tail_my), s_z == i_my),
                 lambda j=j: rdma(opp_base, 3 * j + s_z,
                                  zsend, zrecv, j, z_nbr).start())

            if j >= 2:
                d = j - 2
                when(p(d, tail_xn),
                     lambda d=d: rdma(opp_base, 3 * d + i_xn,
                                      xsend, xrecv, d, x_nbr).wait_recv())
                when(land(p(d, tail_xn), s_z == i_xn),
                     lambda d=d: rdma(opp_base, 3 * d + s_z,
                                      zsend, zrecv, d, z_nbr).start())

        for d in (J - 2, J - 1):
            when(p(d, tail_xn),
                 lambda d=d: rdma(opp_base, 3 * d + i_xn,
                                  xsend, xrecv, d, x_nbr).wait_recv())
            when(land(p(d, tail_xn), s_z == i_xn),
                 lambda d=d: rdma(opp_base, 3 * d + s_z,
                                  zsend, zrecv, d, z_nbr).start())
        for j in range(J):
            when(p(j, tail_miss),
                 lambda j=j: rdma(opp_base, 3 * j + i_miss,
                                  zsend, zrecv, j, z_nbr).wait_recv())
        for j in range(J):
            when(p(j, tail_my),
                 lambda j=j: rdma(own_base, 3 * j + i_my, ysend, yrecv, j,
                                  y_nbr, src=mine_buf.at[mine_rows(j), :]).wait_send())
            when(p(j, tail_my),
                 lambda j=j: rdma(opp_base, 3 * j + i_my,
                                  xsend, xrecv, j, x_nbr).wait_send())
            when(p(j, tail_zs),
                 lambda j=j: rdma(opp_base, 3 * j + s_z,
                                  zsend, zrecv, j, z_nbr).wait_send())
            when(seq[j][1],
                 lambda j=j: pltpu.make_async_copy(
                     mine_buf.at[mine_rows(j), :],
                     out_ref.at[mine_rows(j), :],
                     msem.at[j],
                 ).wait())
        for slot in range(2):
            if stage_uses[0] > slot:
                pltpu.make_async_copy(
                    stage.at[slot], stage.at[slot], store_sems.at[slot]
                ).wait()

    return pl.pallas_call(
        body,
        out_shape=jax.ShapeDtypeStruct((2 * m_per, n), jnp.bfloat16),
        in_specs=[pl.BlockSpec(memory_space=pl.ANY)],
        out_specs=pl.BlockSpec(memory_space=pl.ANY),
        scratch_shapes=[
            pltpu.VMEM((J * CH, n), jnp.bfloat16),
            pltpu.VMEM((2, CH, n), jnp.float32),
            pltpu.VMEM((2, CH, n), jnp.bfloat16),
            pltpu.SemaphoreType.DMA((2,)),
            pltpu.SemaphoreType.DMA((2,)),
            pltpu.SemaphoreType.DMA((J,)),
            pltpu.SemaphoreType.DMA((J,)),
            pltpu.SemaphoreType.DMA((J,)),
            pltpu.SemaphoreType.DMA((J,)),
            pltpu.SemaphoreType.DMA((J,)),
            pltpu.SemaphoreType.DMA((J,)),
            pltpu.SemaphoreType.DMA((J,)),
        ],
        compiler_params=pltpu.CompilerParams(
            collective_id=0,
            vmem_limit_bytes=56 * 1024 * 1024,
        ),
    )(x)
